# baseline (speedup 1.0000x reference)
import functools

import jax
import jax.numpy as jnp
from jax import lax
from jax.experimental import pallas as pl
from jax.experimental.pallas import tpu as pltpu
from jax.experimental.pallas import tpu_sc as plsc

N = 100000
D_IN = 16
COL0 = 48
NUM_SEG = 1024
NW = 32
CHUNK = 3136
LAST_CNT = N - (NW - 1) * CHUNK
G_FULL = CHUNK // 16
G_LAST = LAST_CNT // 16


def _sc_body(x_hbm, data_hbm, w1_hbm, b1_hbm, w2_hbm, b2_hbm, out_hbm,
             xv, idv, bins, partial, w1v, b1v, w2v, b2v):
    cid = lax.axis_index("c")
    sid = lax.axis_index("s")
    wid = sid * 2 + cid
    base = wid * CHUNK
    is_last = wid == NW - 1
    ngroups = jnp.where(is_last, G_LAST, G_FULL)

    pltpu.sync_copy(w1_hbm, w1v)
    pltpu.sync_copy(b1_hbm, b1v)
    pltpu.sync_copy(w2_hbm, w2v)
    pltpu.sync_copy(b2_hbm, b2v)

    @pl.when(jnp.logical_not(is_last))
    def _():
        pltpu.sync_copy(x_hbm.at[pl.ds(base, CHUNK), pl.ds(COL0, D_IN)], xv)
        pltpu.sync_copy(data_hbm.at[pl.ds(base, CHUNK)], idv)

    @pl.when(is_last)
    def _():
        pltpu.sync_copy(x_hbm.at[pl.ds(base, LAST_CNT), pl.ds(COL0, D_IN)],
                        xv.at[pl.ds(0, LAST_CNT)])
        pltpu.sync_copy(data_hbm.at[pl.ds(base, LAST_CNT)],
                        idv.at[pl.ds(0, LAST_CNT)])

    lanes = lax.iota(jnp.int32, 16)
    zero16 = jnp.zeros((16,), jnp.float32)

    def _zero_row(r):
        def _z(j, _):
            bins[r, pl.ds(j * 16, 16)] = zero16
            return 0
        lax.fori_loop(0, NUM_SEG // 16, _z, 0)
    for r in range(16):
        _zero_row(r)

    w1rows = [w1v[j, :] for j in range(16)]
    b1row = b1v[:]
    w2row = w2v[:]
    w1s = [[w1rows[j][k] for k in range(16)] for j in range(16)]
    b1s = [b1row[j] for j in range(16)]
    w2s = [w2row[j] for j in range(16)]
    b2s = b2v[:][0]

    def _group(g, _):
        row0 = g * 16
        rows = row0 + lanes
        s = [plsc.load_gather(xv, [rows, jnp.full((16,), k, jnp.int32)])
             for k in range(16)]
        e = jnp.full((16,), b2s, jnp.float32)
        for j in range(16):
            h = jnp.full((16,), b1s[j], jnp.float32)
            for k in range(16):
                h = h + s[k] * w1s[j][k]
            sig = 1.0 / (1.0 + jnp.exp(-h))
            e = e + (h * sig) * w2s[j]
        ids = idv[pl.ds(row0, 16)]
        plsc.addupdate_scatter(bins, [lanes, ids], e)
        return 0

    lax.fori_loop(0, ngroups, _group, 0)

    def _red(gi, _):
        c0 = gi * 16
        acc = bins[0, pl.ds(c0, 16)]
        for r in range(1, 16):
            acc = acc + bins[r, pl.ds(c0, 16)]
        partial[pl.ds(c0, 16)] = acc
        return 0

    lax.fori_loop(0, NUM_SEG // 16, _red, 0)

    pltpu.sync_copy(partial, out_hbm.at[wid])


@functools.partial(
    pl.kernel,
    mesh=plsc.VectorSubcoreMesh(core_axis_name="c", subcore_axis_name="s"),
    out_type=jax.ShapeDtypeStruct((NW, NUM_SEG), jnp.float32),
    scratch_types=[
        pltpu.VMEM((CHUNK, D_IN), jnp.float32),
        pltpu.VMEM((CHUNK,), jnp.int32),
        pltpu.VMEM((16, NUM_SEG), jnp.float32),
        pltpu.VMEM((NUM_SEG,), jnp.float32),
        pltpu.VMEM((16, 16), jnp.float32),
        pltpu.VMEM((16,), jnp.float32),
        pltpu.VMEM((16,), jnp.float32),
        pltpu.VMEM((16,), jnp.float32),
    ],
    compiler_params=pltpu.CompilerParams(use_tc_tiling_on_sc=False,
                                         needs_layout_passes=False),
)
def _sc_energy(x_hbm, data_hbm, w1_hbm, b1_hbm, w2_hbm, b2_hbm, out_hbm,
               xv, idv, bins, partial, w1v, b1v, w2v, b2v):
    _sc_body(x_hbm, data_hbm, w1_hbm, b1_hbm, w2_hbm, b2_hbm, out_hbm,
             xv, idv, bins, partial, w1v, b1v, w2v, b2v)


def _combine_body(p_ref, o_ref):
    o_ref[...] = jnp.sum(p_ref[...], axis=0, keepdims=True)


def _combine(parts):
    return pl.pallas_call(
        _combine_body,
        out_shape=jax.ShapeDtypeStruct((1, NUM_SEG), jnp.float32),
        in_specs=[pl.BlockSpec((NW, NUM_SEG), lambda: (0, 0))],
        out_specs=pl.BlockSpec((1, NUM_SEG), lambda: (0, 0)),
    )(parts)


def kernel(x, data, pos, W1, b1, W2, b2):
    data_i = data.astype(jnp.int32)
    w1 = W1.astype(jnp.float32)
    b1v = b1.astype(jnp.float32)
    w2 = jnp.reshape(W2, (16,)).astype(jnp.float32)
    b2v = jnp.broadcast_to(b2.astype(jnp.float32), (16,))
    parts = _sc_energy(x, data_i, w1, b1v, w2, b2v)
    E = _combine(parts).reshape(NUM_SEG, 1)
    F = jnp.zeros((N, 3), jnp.float32)
    return (E, F)


# speedup vs baseline: 1.7629x; 1.7629x over previous
import functools

import jax
import jax.numpy as jnp
from jax import lax
from jax.experimental import pallas as pl
from jax.experimental.pallas import tpu as pltpu
from jax.experimental.pallas import tpu_sc as plsc

N = 100000
NUM_SEG = 1024
PACK = 8
ROWS = N // PACK
DPACK = 64 * PACK
HPACK = 16 * PACK
BLK = 2048

NWORK = 16
PER_W = 6256
LAST_CNT = N - (NWORK - 1) * PER_W
G_FULL = PER_W // 16
G_LAST = LAST_CNT // 16
SEG_PER_W = NUM_SEG // NWORK


def _mlp_body(xr_ref, wa_ref, b1_ref, w2_ref, b2_ref, o_ref):
    h = jnp.dot(xr_ref[...], wa_ref[...],
                preferred_element_type=jnp.float32) + b1_ref[...]
    sil = h * (1.0 / (1.0 + jnp.exp(-h)))
    o_ref[...] = jnp.dot(sil, w2_ref[...],
                         preferred_element_type=jnp.float32) + b2_ref[...]


def _mlp(xr, wa, b1t, w2blk, b2t):
    grid = (ROWS + BLK - 1) // BLK
    return pl.pallas_call(
        _mlp_body,
        grid=(grid,),
        in_specs=[
            pl.BlockSpec((BLK, DPACK), lambda i: (i, 0)),
            pl.BlockSpec((DPACK, HPACK), lambda i: (0, 0)),
            pl.BlockSpec((1, HPACK), lambda i: (0, 0)),
            pl.BlockSpec((HPACK, PACK), lambda i: (0, 0)),
            pl.BlockSpec((1, PACK), lambda i: (0, 0)),
        ],
        out_specs=pl.BlockSpec((BLK, PACK), lambda i: (i, 0)),
        out_shape=jax.ShapeDtypeStruct((ROWS, PACK), jnp.float32),
    )(xr, wa, b1t, w2blk, b2t)


def _seg_body(e_hbm, data_hbm, out_hbm, ev, idv, bins, partial, red, seg_out,
              shared, sem):
    sid = lax.axis_index("s")
    base = sid * PER_W
    is_last = sid == NWORK - 1
    ngroups = jnp.where(is_last, G_LAST, G_FULL)

    cp_e = pltpu.make_async_copy(e_hbm.at[pl.ds(base, PER_W)], ev, sem)
    cp_i = pltpu.make_async_copy(data_hbm.at[pl.ds(base, PER_W)], idv, sem)
    cp_e_l = pltpu.make_async_copy(e_hbm.at[pl.ds(base, LAST_CNT)],
                                   ev.at[pl.ds(0, LAST_CNT)], sem)
    cp_i_l = pltpu.make_async_copy(data_hbm.at[pl.ds(base, LAST_CNT)],
                                   idv.at[pl.ds(0, LAST_CNT)], sem)

    @pl.when(jnp.logical_not(is_last))
    def _():
        cp_e.start()
        cp_i.start()

    @pl.when(is_last)
    def _():
        cp_e_l.start()
        cp_i_l.start()

    lanes = lax.iota(jnp.int32, 16)
    zero16 = jnp.zeros((16,), jnp.float32)

    def _z(j, _):
        for r in range(16):
            bins[r, pl.ds(j * 16, 16)] = zero16
        return 0
    lax.fori_loop(0, NUM_SEG // 16, _z, 0)

    @pl.when(jnp.logical_not(is_last))
    def _():
        cp_e.wait()
        cp_i.wait()

    @pl.when(is_last)
    def _():
        cp_e_l.wait()
        cp_i_l.wait()

    def _group(g, _):
        row0 = g * 16
        e = ev[pl.ds(row0, 16)]
        ids = idv[pl.ds(row0, 16)]
        plsc.addupdate_scatter(bins, [lanes, ids], e)
        return 0

    lax.fori_loop(0, ngroups, _group, 0)

    def _red(gi, _):
        c0 = gi * 16
        acc = bins[0, pl.ds(c0, 16)]
        for r in range(1, 16):
            acc = acc + bins[r, pl.ds(c0, 16)]
        partial[pl.ds(c0, 16)] = acc
        return 0
    lax.fori_loop(0, NUM_SEG // 16, _red, 0)

    pltpu.sync_copy(partial, shared.at[sid])
    plsc.subcore_barrier()
    c0 = sid * SEG_PER_W
    pltpu.sync_copy(shared.at[:, pl.ds(c0, SEG_PER_W)], red)
    for j in range(SEG_PER_W // 16):
        acc = red[0, pl.ds(j * 16, 16)]
        for r in range(1, 16):
            acc = acc + red[r, pl.ds(j * 16, 16)]
        seg_out[pl.ds(j * 16, 16)] = acc
    pltpu.sync_copy(seg_out, out_hbm.at[pl.ds(c0, SEG_PER_W)])


@functools.partial(
    pl.kernel,
    mesh=plsc.VectorSubcoreMesh(core_axis_name="c", subcore_axis_name="s",
                                num_cores=1),
    out_type=jax.ShapeDtypeStruct((NUM_SEG,), jnp.float32),
    scratch_types=[
        pltpu.VMEM((PER_W,), jnp.float32),
        pltpu.VMEM((PER_W,), jnp.int32),
        pltpu.VMEM((16, NUM_SEG), jnp.float32),
        pltpu.VMEM((NUM_SEG,), jnp.float32),
        pltpu.VMEM((NWORK, SEG_PER_W), jnp.float32),
        pltpu.VMEM((SEG_PER_W,), jnp.float32),
        pltpu.VMEM_SHARED((NWORK, NUM_SEG), jnp.float32),
        pltpu.SemaphoreType.DMA,
    ],
    compiler_params=pltpu.CompilerParams(use_tc_tiling_on_sc=False,
                                         needs_layout_passes=False),
)
def _sc_segsum(e_hbm, data_hbm, out_hbm, ev, idv, bins, partial, red, seg_out,
               shared, sem):
    _seg_body(e_hbm, data_hbm, out_hbm, ev, idv, bins, partial, red, seg_out,
              shared, sem)


def kernel(x, data, pos, W1, b1, W2, b2):
    data_i = data.astype(jnp.int32)
    w1blk = jnp.zeros((64, 16), jnp.float32).at[48:64, :].set(
        W1.T.astype(jnp.float32))
    eye = jnp.eye(PACK, dtype=jnp.float32)
    wa = jnp.kron(eye, w1blk)
    b1t = jnp.tile(b1.astype(jnp.float32), PACK).reshape(1, HPACK)
    w2blk = jnp.kron(eye, W2.astype(jnp.float32).reshape(16, 1))
    b2t = jnp.broadcast_to(b2.astype(jnp.float32), (1, PACK))

    xr = x.reshape(ROWS, DPACK)
    e = _mlp(xr, wa, b1t, w2blk, b2t).reshape(N)
    E = _sc_segsum(e, data_i).reshape(NUM_SEG, 1)
    F = jnp.zeros((N, 3), jnp.float32)
    return (E, F)
